# baseline (speedup 1.0000x reference)
import functools

import jax
import jax.numpy as jnp
from jax import lax
from jax.experimental import pallas as pl
from jax.experimental.pallas import tpu as pltpu
from jax.experimental.pallas import tpu_sc as plsc

_BATCH = 4096
_SEQ = 200
_HIDDEN = 64
_VOCAB = 1000000
_PADW = 128
_TOTAL = _BATCH * _SEQ
_NW = 32
_CHUNK = 64
_NCHUNK = _TOTAL // (_NW * _CHUNK)
_K = 5
_NSETS = _NCHUNK // _K
_PAIRS = _NSETS // 2


def _make_lookup():
    mesh = plsc.VectorSubcoreMesh(core_axis_name="c", subcore_axis_name="s")

    @functools.partial(
        pl.kernel,
        mesh=mesh,
        out_type=jax.ShapeDtypeStruct((_TOTAL, _PADW), jnp.float32),
        scratch_types=[
            pltpu.VMEM((_NCHUNK, _CHUNK), jnp.int32),
            pltpu.VMEM((2, _K, _CHUNK, _PADW), jnp.float32),
            pltpu.SemaphoreType.DMA,
            pltpu.SemaphoreType.DMA,
            pltpu.SemaphoreType.DMA,
            pltpu.SemaphoreType.DMA,
        ],
        compiler_params=pltpu.CompilerParams(
            use_tc_tiling_on_sc=False, needs_layout_passes=False),
    )
    def lookup(table_hbm, idx_hbm, out_hbm, idx_v, rows_v, g0s, g1s, w0s,
               w1s):
        wid = lax.axis_index("s") * 2 + lax.axis_index("c")
        chunk0 = wid * _NCHUNK
        gsems = (g0s, g1s)
        wsems = (w0s, w1s)

        pltpu.sync_copy(idx_hbm.at[pl.ds(chunk0, _NCHUNK)], idx_v)

        def fire_gathers(s, p):
            for b in range(_K):
                pltpu.async_copy(
                    table_hbm.at[idx_v.at[s * _K + b]],
                    rows_v.at[p].at[b],
                    gsems[p],
                )

        def drain(sem, dst_vmem):
            for b in range(_K):
                if dst_vmem:
                    pltpu.make_async_copy(
                        table_hbm.at[pl.ds(0, _CHUNK)],
                        rows_v.at[0].at[b],
                        sem,
                    ).wait()
                else:
                    pltpu.make_async_copy(
                        rows_v.at[0].at[b],
                        out_hbm.at[pl.ds(0, _CHUNK)],
                        sem,
                    ).wait()

        def fire_writebacks(s, p):
            for b in range(_K):
                ga = (chunk0 + s * _K + b) * _CHUNK
                pltpu.async_copy(
                    rows_v.at[p].at[b],
                    out_hbm.at[pl.ds(ga, _CHUNK)],
                    wsems[p],
                )

        fire_gathers(0, 0)

        def pair(t, carry):
            for p in range(2):
                s = 2 * t + p
                if p == 0:
                    @pl.when(t > 0)
                    def _():
                        drain(wsems[1], False)
                else:
                    drain(wsems[0], False)
                if p == 0:
                    fire_gathers(s + 1, 1)
                else:
                    @pl.when(t < _PAIRS - 1)
                    def _():
                        fire_gathers(s + 1, 0)
                drain(gsems[p], True)
                fire_writebacks(s, p)
            return carry

        lax.fori_loop(0, _PAIRS, pair, 0)

        drain(wsems[1], False)

    return lookup


_lookup = _make_lookup()


def kernel(item_seqs, emb):
    table3 = jnp.pad(emb, ((0, 0), (0, _PADW - _HIDDEN)))
    flat_idx = item_seqs.reshape(_TOTAL // _CHUNK, _CHUNK)
    out = _lookup(table3, flat_idx)
    return out.reshape(_BATCH, _SEQ, _PADW)[:, :, :_HIDDEN]
